# baseline (speedup 1.0000x reference)
import functools

import jax
import jax.numpy as jnp
from jax import lax
from jax.experimental import pallas as pl
from jax.experimental.pallas import tpu as pltpu
from jax.experimental.pallas import tpu_sc as plsc

_NC = 2
_NS = 16
_NW = _NC * _NS
_L = 16
_D = 128
_C = 80
_NBUF = 2


def _normalize_body(z_ref, out_ref):
    x = z_ref[...]
    ss = jnp.sum(x * x, axis=-1, keepdims=True)
    out_ref[...] = x / jnp.maximum(jnp.sqrt(ss), 1e-12)


def _make_edge_dot(n_edges: int):
    ew = n_edges // _NW
    n_chunks = ew // _C
    mesh = plsc.VectorSubcoreMesh(core_axis_name="c", subcore_axis_name="s")

    @functools.partial(
        pl.kernel,
        mesh=mesh,
        out_type=jax.ShapeDtypeStruct((_NW, n_chunks, _C), jnp.float32),
        compiler_params=pltpu.CompilerParams(
            needs_layout_passes=False, use_tc_tiling_on_sc=False
        ),
        scratch_types=[
            pltpu.VMEM((n_chunks, _C), jnp.int32),
            pltpu.VMEM((n_chunks, _C), jnp.int32),
            pltpu.VMEM((n_chunks, _C), jnp.float32),
            [
                dict(
                    rows_u=pltpu.VMEM((_C, _D // 2), jnp.int32),
                    rows_v=pltpu.VMEM((_C, _D // 2), jnp.int32),
                    sem=pltpu.SemaphoreType.DMA,
                )
                for _ in range(_NBUF)
            ],
        ],
    )
    def edge_dot(zn, u, v, out, idx_u, idx_v, oc, bufs):
        wid = lax.axis_index("s") * _NC + lax.axis_index("c")

        pltpu.sync_copy(u.at[wid], idx_u)
        pltpu.sync_copy(v.at[wid], idx_v)

        def issue(t, b):
            pltpu.async_copy(zn.at[idx_u.at[t]], b["rows_u"], b["sem"])
            pltpu.async_copy(zn.at[idx_v.at[t]], b["rows_v"], b["sem"])

        def drain(t, b):
            pltpu.make_async_copy(zn.at[idx_u.at[t]], b["rows_u"], b["sem"]).wait()
            pltpu.make_async_copy(zn.at[idx_v.at[t]], b["rows_v"], b["sem"]).wait()

        lanes = lax.iota(jnp.int32, _L)

        def compute(t, b):
            ru, rv = b["rows_u"], b["rows_v"]

            def edge_dot16(e):
                acc_a = None
                acc_b = None
                for k in range(_D // (2 * _L)):
                    xu = plsc.bitcast(ru[e, pl.ds(k * _L, _L)], jnp.bfloat16)
                    xv = plsc.bitcast(rv[e, pl.ds(k * _L, _L)], jnp.bfloat16)
                    pa, pb = plsc.unpack(xu * xv, format=plsc.PackFormat.INTERLEAVED)
                    acc_a = pa if acc_a is None else acc_a + pa
                    acc_b = pb if acc_b is None else acc_b + pb
                return acc_a + acc_b

            def group_body(g, carry):
                r0 = pl.multiple_of(g * _L, _L)
                res = jnp.zeros((_L,), jnp.float32)
                for j in range(_L):
                    res = jnp.where(lanes == j, jnp.sum(edge_dot16(r0 + j)), res)
                oc[t, pl.ds(r0, _L)] = res
                return carry

            lax.fori_loop(0, _C // _L, group_body, 0)

        for p in range(_NBUF):
            issue(p, bufs[p])

        def pipe_body(k, carry):
            t0 = k * _NBUF
            for p in range(_NBUF):
                t = t0 + p
                drain(t, bufs[p])
                compute(t, bufs[p])

                @pl.when(t + _NBUF < n_chunks)
                def _():
                    issue(t + _NBUF, bufs[p])

            return carry

        lax.fori_loop(0, n_chunks // _NBUF, pipe_body, 0)
        for p in range(n_chunks % _NBUF):
            t = (n_chunks // _NBUF) * _NBUF + p
            drain(t, bufs[p])
            compute(t, bufs[p])
        pltpu.sync_copy(oc, out.at[wid])

    return edge_dot


def kernel(z, edge_index):
    n, d = z.shape
    assert d == _D
    zn = pl.pallas_call(
        _normalize_body,
        out_shape=jax.ShapeDtypeStruct((n, d), jnp.float32),
    )(z)
    znp = jax.lax.bitcast_convert_type(
        zn.astype(jnp.bfloat16).reshape(n, d // 2, 2), jnp.int32
    )
    u = edge_index[0].astype(jnp.int32)
    v = edge_index[1].astype(jnp.int32)
    n_edges = u.shape[0]
    assert n_edges % (_NW * _C) == 0
    n_chunks = n_edges // (_NW * _C)
    u3 = u.reshape(_NW, n_chunks, _C)
    v3 = v.reshape(_NW, n_chunks, _C)
    out = _make_edge_dot(n_edges)(znp, u3, v3)
    return out.reshape(n_edges)


# speedup vs baseline: 1.2495x; 1.2495x over previous
import functools

import jax
import jax.numpy as jnp
from jax import lax
from jax.experimental import pallas as pl
from jax.experimental.pallas import tpu as pltpu
from jax.experimental.pallas import tpu_sc as plsc

_NC = 2
_NS = 16
_NW = _NC * _NS
_L = 16
_D = 128
_C = 200
_NBUF = 2


def _normalize_body(z_ref, out_ref):
    x = z_ref[...]
    ss = jnp.sum(x * x, axis=-1, keepdims=True)
    out_ref[...] = x / jnp.maximum(jnp.sqrt(ss), 1e-12)


def _make_edge_dot(n_edges: int):
    ew = n_edges // _NW
    n_chunks = ew // _C
    mesh = plsc.VectorSubcoreMesh(core_axis_name="c", subcore_axis_name="s")

    @functools.partial(
        pl.kernel,
        mesh=mesh,
        out_type=jax.ShapeDtypeStruct((_NW, n_chunks, _C), jnp.float32),
        compiler_params=pltpu.CompilerParams(
            needs_layout_passes=False, use_tc_tiling_on_sc=False
        ),
        scratch_types=[
            pltpu.VMEM_SHARED((10000, _D // 2), jnp.int32),
            pltpu.VMEM((n_chunks, _C), jnp.int32),
            pltpu.VMEM((n_chunks, _C), jnp.int32),
            pltpu.VMEM((n_chunks, _C), jnp.float32),
            [
                dict(
                    rows_u=pltpu.VMEM((_C, _D // 2), jnp.int32),
                    rows_v=pltpu.VMEM((_C, _D // 2), jnp.int32),
                    sem=pltpu.SemaphoreType.DMA,
                )
                for _ in range(_NBUF)
            ],
        ],
    )
    def edge_dot(zn, u, v, out, tab, idx_u, idx_v, oc, bufs):
        wid = lax.axis_index("s") * _NC + lax.axis_index("c")

        @pl.when(lax.axis_index("s") == 0)
        def _():
            pltpu.sync_copy(zn, tab)

        pltpu.sync_copy(u.at[wid], idx_u)
        pltpu.sync_copy(v.at[wid], idx_v)
        plsc.subcore_barrier()

        def issue(t, b):
            pltpu.async_copy(tab.at[idx_u.at[t]], b["rows_u"], b["sem"])
            pltpu.async_copy(tab.at[idx_v.at[t]], b["rows_v"], b["sem"])

        def drain(t, b):
            pltpu.make_async_copy(tab.at[idx_u.at[t]], b["rows_u"], b["sem"]).wait()
            pltpu.make_async_copy(tab.at[idx_v.at[t]], b["rows_v"], b["sem"]).wait()

        lanes = lax.iota(jnp.int32, _L)

        def compute(t, b):
            ru, rv = b["rows_u"], b["rows_v"]

            def edge_dot16(e):
                acc_a = None
                acc_b = None
                for k in range(_D // (2 * _L)):
                    xu = plsc.bitcast(ru[e, pl.ds(k * _L, _L)], jnp.bfloat16)
                    xv = plsc.bitcast(rv[e, pl.ds(k * _L, _L)], jnp.bfloat16)
                    pa, pb = plsc.unpack(xu * xv, format=plsc.PackFormat.INTERLEAVED)
                    acc_a = pa if acc_a is None else acc_a + pa
                    acc_b = pb if acc_b is None else acc_b + pb
                return acc_a + acc_b

            def group_body(g, carry):
                r0 = pl.multiple_of(g * _L, _L)
                res = jnp.zeros((_L,), jnp.float32)
                for j in range(_L):
                    res = jnp.where(lanes == j, jnp.sum(edge_dot16(r0 + j)), res)
                oc[t, pl.ds(r0, _L)] = res
                return carry

            lax.fori_loop(0, _C // _L, group_body, 0)

        for p in range(_NBUF):
            issue(p, bufs[p])

        def pipe_body(k, carry):
            t0 = k * _NBUF
            for p in range(_NBUF):
                t = t0 + p
                drain(t, bufs[p])
                compute(t, bufs[p])

                @pl.when(t + _NBUF < n_chunks)
                def _():
                    issue(t + _NBUF, bufs[p])

            return carry

        lax.fori_loop(0, n_chunks // _NBUF, pipe_body, 0)
        for p in range(n_chunks % _NBUF):
            t = (n_chunks // _NBUF) * _NBUF + p
            drain(t, bufs[p])
            compute(t, bufs[p])
        pltpu.sync_copy(oc, out.at[wid])

    return edge_dot


def kernel(z, edge_index):
    n, d = z.shape
    assert d == _D
    zn = pl.pallas_call(
        _normalize_body,
        out_shape=jax.ShapeDtypeStruct((n, d), jnp.float32),
    )(z)
    znp = jax.lax.bitcast_convert_type(
        zn.astype(jnp.bfloat16).reshape(n, d // 2, 2), jnp.int32
    )
    u = edge_index[0].astype(jnp.int32)
    v = edge_index[1].astype(jnp.int32)
    n_edges = u.shape[0]
    assert n_edges % (_NW * _C) == 0
    n_chunks = n_edges // (_NW * _C)
    u3 = u.reshape(_NW, n_chunks, _C)
    v3 = v.reshape(_NW, n_chunks, _C)
    out = _make_edge_dot(n_edges)(znp, u3, v3)
    return out.reshape(n_edges)
